# Initial kernel scaffold; baseline (speedup 1.0000x reference)
import jax
import jax.numpy as jnp
from jax.experimental import pallas as pl


def kernel(x, edge_h, bf, edge_index, W_fgn1, b_fgn1, W_fgn2, b_fgn2, W_ib1, b_ib1, W_ib2, b_ib2):
    raise NotImplementedError("write your pallas kernel here")


# speedup vs baseline: 1.4731x; 1.4731x over previous
import functools

import jax
import jax.numpy as jnp
from jax import lax
from jax.experimental import pallas as pl
from jax.experimental.pallas import tpu as pltpu
from jax.experimental.pallas import tpu_sc as plsc

_LOG2 = 0.6931471805599453

_NC = 2
_NS = 16
_LANES = 16


def _ssp(v):
    return jnp.maximum(v, 0.0) + jnp.log1p(jnp.exp(-jnp.abs(v))) - _LOG2


def _edge_mlp_body(bf_ref, eh_in_ref, w1_ref, b1_ref, w2_ref, b2_ref, out_ref):
    h = jnp.dot(bf_ref[...], w1_ref[...], preferred_element_type=jnp.float32)
    h = _ssp(h + b1_ref[...])
    h = jnp.dot(h, w2_ref[...], preferred_element_type=jnp.float32)
    h = _ssp(h + b2_ref[...]) * eh_in_ref[...]
    out_ref[0] = h[:, :32]
    out_ref[1] = h[:, 32:]


def _edge_mlp(bf, edge_h, w1, b1, w2, b2, block_e=2000):
    e, r = bf.shape
    f = edge_h.shape[1]
    grid = e // block_e
    return pl.pallas_call(
        _edge_mlp_body,
        grid=(grid,),
        in_specs=[
            pl.BlockSpec((block_e, r), lambda i: (i, 0)),
            pl.BlockSpec((block_e, f), lambda i: (i, 0)),
            pl.BlockSpec((r, f), lambda i: (0, 0)),
            pl.BlockSpec((1, f), lambda i: (0, 0)),
            pl.BlockSpec((f, f), lambda i: (0, 0)),
            pl.BlockSpec((1, f), lambda i: (0, 0)),
        ],
        out_specs=pl.BlockSpec((2, block_e, f // 2), lambda i: (0, i, 0)),
        out_shape=jax.ShapeDtypeStruct((2, e, f // 2), jnp.float32),
        compiler_params=pltpu.CompilerParams(
            dimension_semantics=("arbitrary",)),
    )(bf, edge_h, w1, b1, w2, b2)


def _make_sc_scatter(n, e, chunk):
    per_sub = e // _NS
    nchunk = per_sub // chunk
    rows_a = ((n // _NS) + 7) // 8 * 8
    rows_last = n - (_NS - 1) * rows_a
    mesh = plsc.VectorSubcoreMesh(core_axis_name="c", subcore_axis_name="s")

    def sc_body(x2_hbm, eh2_hbm, src_hbm, dst_hbm, zer_hbm, zc_hbm, one_hbm,
                summed_hbm, cnt_hbm,
                srcv, dstv, ehv, xv, onesv, acc, cntacc, sem):
        c = lax.axis_index("c")
        s = lax.axis_index("s")
        row0 = s * rows_a

        @pl.when(s < _NS - 1)
        def _():
            pltpu.sync_copy(zer_hbm, acc.at[pl.ds(row0, rows_a)])
            pltpu.sync_copy(zc_hbm, cntacc.at[pl.ds(row0, rows_a)])

        @pl.when(s == _NS - 1)
        def _():
            pltpu.sync_copy(zer_hbm.at[pl.ds(0, rows_last)],
                            acc.at[pl.ds(row0, rows_last)])
            pltpu.sync_copy(zc_hbm.at[pl.ds(0, rows_last)],
                            cntacc.at[pl.ds(row0, rows_last)])

        pltpu.sync_copy(one_hbm, onesv)
        plsc.subcore_barrier()

        coff = c * n

        def chunk_body(j, _):
            e0 = s * per_sub + j * chunk
            pltpu.sync_copy(src_hbm.at[pl.ds(e0, chunk)], srcv)
            pltpu.sync_copy(dst_hbm.at[pl.ds(e0, chunk)], dstv)
            pltpu.sync_copy(eh2_hbm.at[c, pl.ds(e0, chunk)], ehv)
            for k in range(chunk // _LANES):
                sl = pl.ds(k * _LANES, _LANES)
                srcv[sl] = srcv[sl] + coff
            pltpu.async_copy(x2_hbm.at[srcv], xv, sem).wait()
            def mul_body(rr, _):
                lo = pl.ds(0, _LANES)
                hi = pl.ds(_LANES, _LANES)
                xv[rr, lo] = xv[rr, lo] * ehv[rr, lo]
                xv[rr, hi] = xv[rr, hi] * ehv[rr, hi]
                return 0

            lax.fori_loop(0, chunk, mul_body, 0)
            pltpu.sync_copy(xv, acc.at[dstv], add=True)

            @pl.when((j % 2) == c)
            def _():
                pltpu.sync_copy(onesv, cntacc.at[dstv], add=True)

            return 0

        lax.fori_loop(0, nchunk, chunk_body, 0)
        plsc.subcore_barrier()

        @pl.when(s < _NS - 1)
        def _():
            pltpu.sync_copy(acc.at[pl.ds(row0, rows_a)],
                            summed_hbm.at[c, pl.ds(row0, rows_a)])
            pltpu.sync_copy(cntacc.at[pl.ds(row0, rows_a)],
                            cnt_hbm.at[c, pl.ds(row0, rows_a)])

        @pl.when(s == _NS - 1)
        def _():
            pltpu.sync_copy(acc.at[pl.ds(row0, rows_last)],
                            summed_hbm.at[c, pl.ds(row0, rows_last)])
            pltpu.sync_copy(cntacc.at[pl.ds(row0, rows_last)],
                            cnt_hbm.at[c, pl.ds(row0, rows_last)])

    return pl.kernel(
        sc_body,
        out_type=(
            jax.ShapeDtypeStruct((_NC, n, 32), jnp.float32),
            jax.ShapeDtypeStruct((_NC, n, 8), jnp.float32),
        ),
        mesh=mesh,
        scratch_types=[
            pltpu.VMEM((chunk,), jnp.int32),
            pltpu.VMEM((chunk,), jnp.int32),
            pltpu.VMEM((chunk, 32), jnp.float32),
            pltpu.VMEM((chunk, 32), jnp.float32),
            pltpu.VMEM((chunk, 8), jnp.float32),
            pltpu.VMEM_SHARED((n, 32), jnp.float32),
            pltpu.VMEM_SHARED((n, 8), jnp.float32),
            pltpu.SemaphoreType.DMA,
        ],
        compiler_params=pltpu.CompilerParams(use_tc_tiling_on_sc=False),
    )


def _node_mlp_body(sum_ref, cnt_ref, w1_ref, b1_ref, w2_ref, b2_ref, out_ref):
    h = jnp.concatenate([sum_ref[0], sum_ref[1]], axis=1)
    cv = cnt_ref[0, :, 0:1] + cnt_ref[1, :, 0:1]
    h = h / jnp.maximum(cv, 1.0)
    h = jnp.dot(h, w1_ref[...], preferred_element_type=jnp.float32)
    h = _ssp(h + b1_ref[...])
    h = jnp.dot(h, w2_ref[...], preferred_element_type=jnp.float32)
    out_ref[...] = _ssp(h + b2_ref[...])


def _node_mlp(summed, cnt, w1, b1, w2, b2, block_n=2000):
    n = summed.shape[1]
    f = w1.shape[0]
    grid = n // block_n
    return pl.pallas_call(
        _node_mlp_body,
        grid=(grid,),
        in_specs=[
            pl.BlockSpec((2, block_n, 32), lambda i: (0, i, 0)),
            pl.BlockSpec((2, block_n, 8), lambda i: (0, i, 0)),
            pl.BlockSpec((f, f), lambda i: (0, 0)),
            pl.BlockSpec((1, f), lambda i: (0, 0)),
            pl.BlockSpec((f, f), lambda i: (0, 0)),
            pl.BlockSpec((1, f), lambda i: (0, 0)),
        ],
        out_specs=pl.BlockSpec((block_n, f), lambda i: (i, 0)),
        out_shape=jax.ShapeDtypeStruct((n, f), jnp.float32),
        compiler_params=pltpu.CompilerParams(
            dimension_semantics=("arbitrary",)),
    )(summed, cnt, w1, b1, w2, b2)


def kernel(x, edge_h, bf, edge_index, W_fgn1, b_fgn1, W_fgn2, b_fgn2,
           W_ib1, b_ib1, W_ib2, b_ib2):
    n, f = x.shape
    e = bf.shape[0]
    chunk = 80

    eh2 = _edge_mlp(bf, edge_h, W_fgn1, b_fgn1.reshape(1, f),
                    W_fgn2, b_fgn2.reshape(1, f))

    x2 = jnp.concatenate([x[:, :32], x[:, 32:]], axis=0)
    src = edge_index[0]
    dst = edge_index[1]
    rows_a = ((n // _NS) + 7) // 8 * 8
    zer = jnp.zeros((rows_a, 32), jnp.float32)
    zc = jnp.zeros((rows_a, 8), jnp.float32)
    one = jnp.zeros((chunk, 8), jnp.float32).at[:, 0].set(1.0)

    sc = _make_sc_scatter(n, e, chunk)
    summed, cnt = sc(x2, eh2, src, dst, zer, zc, one)

    return _node_mlp(summed, cnt, W_ib1, b_ib1.reshape(1, f),
                     W_ib2, b_ib2.reshape(1, f))


# speedup vs baseline: 1.5567x; 1.0567x over previous
import functools

import jax
import jax.numpy as jnp
from jax import lax
from jax.experimental import pallas as pl
from jax.experimental.pallas import tpu as pltpu
from jax.experimental.pallas import tpu_sc as plsc

_LOG2 = 0.6931471805599453

_NC = 2
_NS = 16
_LANES = 16
_C = 128


def _ssp(v):
    r = jnp.log(1.0 + jnp.exp(jnp.minimum(v, 60.0)))
    return jnp.maximum(r, v) - _LOG2


def _edge_mlp_body(bf_ref, eh_in_ref, w1_ref, b1_ref, w2_ref, b2_ref, out_ref):
    h = jnp.dot(bf_ref[...], w1_ref[...], preferred_element_type=jnp.float32)
    h = _ssp(h + b1_ref[...])
    h = jnp.dot(h, w2_ref[...], preferred_element_type=jnp.float32)
    h = _ssp(h + b2_ref[...]) * eh_in_ref[...]
    out_ref[0] = h[:, :32]
    out_ref[1] = h[:, 32:]


def _edge_mlp(bf, edge_h, w1, b1, w2, b2, block_e=3200):
    e, r = bf.shape
    f = edge_h.shape[1]
    grid = e // block_e
    return pl.pallas_call(
        _edge_mlp_body,
        grid=(grid,),
        in_specs=[
            pl.BlockSpec((block_e, r), lambda i: (i, 0)),
            pl.BlockSpec((block_e, f), lambda i: (i, 0)),
            pl.BlockSpec((r, f), lambda i: (0, 0)),
            pl.BlockSpec((1, f), lambda i: (0, 0)),
            pl.BlockSpec((f, f), lambda i: (0, 0)),
            pl.BlockSpec((1, f), lambda i: (0, 0)),
        ],
        out_specs=pl.BlockSpec((2, block_e, 32), lambda i: (0, i, 0)),
        out_shape=jax.ShapeDtypeStruct((2, e, 32), jnp.float32),
        compiler_params=pltpu.CompilerParams(
            dimension_semantics=("arbitrary",)),
    )(bf, edge_h, w1, b1, w2, b2)


def _make_sc_scatter(n, e, chunk=80):
    per_sub = e // _NS
    nchunk = per_sub // chunk
    rows_a = ((n // _NS) + 7) // 8 * 8
    rows_last = n - (_NS - 1) * rows_a
    mesh = plsc.VectorSubcoreMesh(core_axis_name="c", subcore_axis_name="s")

    def sc_body(x2_hbm, eh2_hbm, src_hbm, dst_hbm, zer_hbm, zc_hbm, one_hbm,
                summed_hbm, cnt_hbm,
                srcv, dstv, ehv, xv, onesv, acc, cntacc, sem):
        c = lax.axis_index("c")
        s = lax.axis_index("s")
        row0 = s * rows_a

        @pl.when(s < _NS - 1)
        def _():
            pltpu.sync_copy(zer_hbm, acc.at[pl.ds(row0, rows_a)])
            pltpu.sync_copy(zc_hbm, cntacc.at[pl.ds(row0, rows_a)])

        @pl.when(s == _NS - 1)
        def _():
            pltpu.sync_copy(zer_hbm.at[pl.ds(0, rows_last)],
                            acc.at[pl.ds(row0, rows_last)])
            pltpu.sync_copy(zc_hbm.at[pl.ds(0, rows_last)],
                            cntacc.at[pl.ds(row0, rows_last)])

        pltpu.sync_copy(one_hbm, onesv)
        plsc.subcore_barrier()

        coff = c * n

        def chunk_body(j, _):
            e0 = s * per_sub + j * chunk
            pltpu.sync_copy(src_hbm.at[pl.ds(e0, chunk)], srcv)
            pltpu.sync_copy(dst_hbm.at[pl.ds(e0, chunk)], dstv)
            pltpu.sync_copy(eh2_hbm.at[c, pl.ds(e0, chunk)], ehv)
            for k in range(chunk // _LANES):
                sl = pl.ds(k * _LANES, _LANES)
                srcv[sl] = srcv[sl] + coff
            pltpu.async_copy(x2_hbm.at[srcv], xv, sem).wait()
            lo = pl.ds(0, _LANES)
            hi = pl.ds(_LANES, _LANES)

            def mul_body(rr, _):
                xv[rr, lo] = xv[rr, lo] * ehv[rr, lo]
                xv[rr, hi] = xv[rr, hi] * ehv[rr, hi]
                return 0

            lax.fori_loop(0, chunk, mul_body, 0)
            pltpu.sync_copy(xv, acc.at[dstv], add=True)

            @pl.when((j % 2) == c)
            def _():
                pltpu.sync_copy(onesv, cntacc.at[dstv], add=True)

            return 0

        lax.fori_loop(0, nchunk, chunk_body, 0)
        plsc.subcore_barrier()

        @pl.when(s < _NS - 1)
        def _():
            pltpu.sync_copy(acc.at[pl.ds(row0, rows_a)],
                            summed_hbm.at[c, pl.ds(row0, rows_a)])
            pltpu.sync_copy(cntacc.at[pl.ds(row0, rows_a)],
                            cnt_hbm.at[c, pl.ds(row0, rows_a)])

        @pl.when(s == _NS - 1)
        def _():
            pltpu.sync_copy(acc.at[pl.ds(row0, rows_last)],
                            summed_hbm.at[c, pl.ds(row0, rows_last)])
            pltpu.sync_copy(cntacc.at[pl.ds(row0, rows_last)],
                            cnt_hbm.at[c, pl.ds(row0, rows_last)])

    return pl.kernel(
        sc_body,
        out_type=(
            jax.ShapeDtypeStruct((_NC, n, 32), jnp.float32),
            jax.ShapeDtypeStruct((_NC, n, 8), jnp.float32),
        ),
        mesh=mesh,
        scratch_types=[
            pltpu.VMEM((chunk,), jnp.int32),
            pltpu.VMEM((chunk,), jnp.int32),
            pltpu.VMEM((chunk, 32), jnp.float32),
            pltpu.VMEM((chunk, 32), jnp.float32),
            pltpu.VMEM((chunk, 8), jnp.float32),
            pltpu.VMEM_SHARED((n, 32), jnp.float32),
            pltpu.VMEM_SHARED((n, 8), jnp.float32),
            pltpu.SemaphoreType.DMA,
        ],
        compiler_params=pltpu.CompilerParams(use_tc_tiling_on_sc=False),
    )


def _node_mlp_body(sum_ref, cnt_ref, w1_ref, b1_ref, w2_ref, b2_ref, out_ref):
    h = jnp.concatenate([sum_ref[0], sum_ref[1]], axis=1)
    cv = cnt_ref[0, :, 0:1] + cnt_ref[1, :, 0:1]
    h = h / jnp.maximum(cv, 1.0)
    h = jnp.dot(h, w1_ref[...], preferred_element_type=jnp.float32)
    h = _ssp(h + b1_ref[...])
    h = jnp.dot(h, w2_ref[...], preferred_element_type=jnp.float32)
    out_ref[...] = _ssp(h + b2_ref[...])


def _node_mlp(summed, cnt, w1, b1, w2, b2, block_n=2000):
    n = summed.shape[1]
    f = w1.shape[0]
    grid = n // block_n
    return pl.pallas_call(
        _node_mlp_body,
        grid=(grid,),
        in_specs=[
            pl.BlockSpec((2, block_n, 32), lambda i: (0, i, 0)),
            pl.BlockSpec((2, block_n, 8), lambda i: (0, i, 0)),
            pl.BlockSpec((f, f), lambda i: (0, 0)),
            pl.BlockSpec((1, f), lambda i: (0, 0)),
            pl.BlockSpec((f, f), lambda i: (0, 0)),
            pl.BlockSpec((1, f), lambda i: (0, 0)),
        ],
        out_specs=pl.BlockSpec((block_n, f), lambda i: (i, 0)),
        out_shape=jax.ShapeDtypeStruct((n, f), jnp.float32),
        compiler_params=pltpu.CompilerParams(
            dimension_semantics=("arbitrary",)),
    )(summed, cnt, w1, b1, w2, b2)


def kernel(x, edge_h, bf, edge_index, W_fgn1, b_fgn1, W_fgn2, b_fgn2,
           W_ib1, b_ib1, W_ib2, b_ib2):
    n, f = x.shape
    e = bf.shape[0]

    eh2 = _edge_mlp(bf, edge_h, W_fgn1, b_fgn1.reshape(1, f),
                    W_fgn2, b_fgn2.reshape(1, f))

    x2 = jnp.concatenate([x[:, :32], x[:, 32:]], axis=0)
    src = edge_index[0]
    dst = edge_index[1]
    chunk = 80
    rows_a = ((n // _NS) + 7) // 8 * 8
    zer = jnp.zeros((rows_a, 32), jnp.float32)
    zc = jnp.zeros((rows_a, 8), jnp.float32)
    one = jnp.zeros((chunk, 8), jnp.float32).at[:, 0].set(1.0)

    sc = _make_sc_scatter(n, e, chunk)
    summed, cnt = sc(x2, eh2, src, dst, zer, zc, one)

    return _node_mlp(summed, cnt, W_ib1, b_ib1.reshape(1, f),
                     W_ib2, b_ib2.reshape(1, f))
